# baseline (speedup 1.0000x reference)
import jax
import jax.numpy as jnp
from jax.experimental import pallas as pl
from jax.experimental.pallas import tpu as pltpu
from jax.experimental.pallas import tpu_sc as plsc

_TAGSET = 48
_T = 50
_START = 48
_STOP = 49
_B = 16
_L = 2048
_CHUNK = 256
_RENORM = 4


def _forward_body(sl_ref, transT_ref, trans_ref, featsF_ref, featsR_ref,
                  out_ref, aF_ref, cF_ref, aB_ref, cB_ref, egF_ref, egB_ref):
    pid = pl.program_id(0)
    trans = trans_ref[...]
    transT = transT_ref[...]
    mrow = jnp.max(transT, axis=0, keepdims=True)
    eT = jnp.exp(transT - mrow)
    mcolB = jnp.max(trans, axis=0, keepdims=True)
    EB = jnp.exp(trans - mcolB)
    emcB = jnp.exp(mcolB)

    @pl.when(pid == 0)
    def _init():
        lane = jax.lax.broadcasted_iota(jnp.int32, (_B, _T), 1)
        aF_ref[...] = jnp.where(lane == _START, 1.0, 0.0)
        cF_ref[...] = jnp.zeros((_B, 1), jnp.float32)
        srow = trans[_STOP:_STOP + 1, :]
        m0 = jnp.max(srow, axis=1, keepdims=True)
        aB_ref[...] = jnp.broadcast_to(jnp.exp(srow - m0), (_B, _T))
        cB_ref[...] = jnp.broadcast_to(m0, (_B, 1))

    egF_ref[...] = jnp.exp(jnp.transpose(featsF_ref[...], (1, 0, 2)) + mrow)
    egB_ref[...] = jnp.exp(jnp.transpose(featsR_ref[...], (1, 0, 2)))
    sl = sl_ref[...]

    def group(g, st):
        aF, cF, aB, cB = st
        for j in range(_RENORM):
            i = g * _RENORM + j
            tF = pid * _CHUNK + i
            tB = _L - 1 - tF
            sF = jax.lax.dot_general(aF, eT, (((1,), (0,)), ((), ())),
                                     preferred_element_type=jnp.float32)
            sF = sF * egF_ref[i]
            aF = jnp.where(tF < sl, sF, aF)
            sB = jax.lax.dot_general(aB * egB_ref[_CHUNK - 1 - i], EB,
                                     (((1,), (0,)), ((), ())),
                                     preferred_element_type=jnp.float32)
            sB = sB * emcB
            aB = jnp.where(tB < sl, sB, aB)
        mF = jnp.max(aF, axis=1, keepdims=True)
        mB = jnp.max(aB, axis=1, keepdims=True)
        return (aF * (1.0 / mF), cF + jnp.log(mF),
                aB * (1.0 / mB), cB + jnp.log(mB))

    aF, cF, aB, cB = jax.lax.fori_loop(
        0, _CHUNK // _RENORM, group,
        (aF_ref[...], cF_ref[...], aB_ref[...], cB_ref[...]))
    aF_ref[...] = aF
    cF_ref[...] = cF
    aB_ref[...] = aB
    cB_ref[...] = cB

    @pl.when(pid == pl.num_programs(0) - 1)
    def _final():
        d = jnp.sum(aF * aB, axis=1, keepdims=True)
        out_ref[...] = cF + cB + jnp.log(d)


_HALF = _L // 2


def _gold_sc_body(sl_hbm, trans_hbm, feats_hbm, tags_hbm, tagsp_hbm, out_hbm,
                  sl_v, trans_v, feats_v, tags_v, tagsp_v, acc_v):
    c = jax.lax.axis_index("c")
    s = jax.lax.axis_index("s")
    wid = s * 2 + c
    b = wid // 2
    h = wid % 2
    t0 = h * _HALF
    pltpu.sync_copy(sl_hbm, sl_v)
    pltpu.sync_copy(trans_hbm, trans_v)
    pltpu.sync_copy(feats_hbm.at[pl.ds((b * _L + t0) * _T, _HALF * _T)], feats_v)
    pltpu.sync_copy(tags_hbm.at[pl.ds(b * _L + t0, _HALF)], tags_v)
    pltpu.sync_copy(tagsp_hbm.at[pl.ds(b * _L + t0, _HALF)], tagsp_v)
    b_vec = jnp.zeros((16,), jnp.int32) + b
    slv = plsc.load_gather(sl_v, [b_vec])
    stop_vec = jnp.full((16,), _STOP, jnp.int32)

    def chunk(i, acc):
        tl = jax.lax.iota(jnp.int32, 16) + i * 16
        tn = tags_v[pl.ds(i * 16, 16)]
        tp = tagsp_v[pl.ds(i * 16, 16)]
        e = plsc.load_gather(feats_v, [tl * _T + tn])
        g = plsc.load_gather(trans_v, [tn * _T + tp])
        st = plsc.load_gather(trans_v, [stop_vec * _T + tn])
        tg = tl + t0
        val = jnp.where(tg < slv, e + g, 0.0)
        val = val + jnp.where(tg == slv - 1, st, 0.0)
        return acc + val

    acc = jax.lax.fori_loop(0, _HALF // 16, chunk,
                            jnp.zeros((16,), jnp.float32))
    acc_v[...] = acc
    pltpu.sync_copy(acc_v, out_hbm.at[wid])


def kernel(feats, tags, seq_lengths, transitions):
    transT = jnp.transpose(transitions, (1, 0))
    sl_col = seq_lengths.reshape(_B, 1)
    tags_prev = jnp.concatenate(
        [jnp.full((_B, 1), _START, dtype=tags.dtype), tags[:, :-1]], axis=1)

    half = _L // 2
    n_blocks = half // _CHUNK
    nb_total = _L // _CHUNK
    fs = pl.pallas_call(
        _forward_body,
        grid=(n_blocks,),
        in_specs=[
            pl.BlockSpec((_B, 1), lambda i: (0, 0)),
            pl.BlockSpec((_T, _T), lambda i: (0, 0)),
            pl.BlockSpec((_T, _T), lambda i: (0, 0)),
            pl.BlockSpec((_B, _CHUNK, _T), lambda i: (0, i, 0)),
            pl.BlockSpec((_B, _CHUNK, _T), lambda i: (0, nb_total - 1 - i, 0)),
        ],
        out_specs=pl.BlockSpec((_B, 1), lambda i: (0, 0)),
        out_shape=jax.ShapeDtypeStruct((_B, 1), jnp.float32),
        scratch_shapes=[pltpu.VMEM((_B, _T), jnp.float32),
                        pltpu.VMEM((_B, 1), jnp.float32),
                        pltpu.VMEM((_B, _T), jnp.float32),
                        pltpu.VMEM((_B, 1), jnp.float32),
                        pltpu.VMEM((_CHUNK, _B, _T), jnp.float32),
                        pltpu.VMEM((_CHUNK, _B, _T), jnp.float32)],
        compiler_params=pltpu.CompilerParams(
            dimension_semantics=("arbitrary",)),
    )(sl_col, transT, transitions, feats, feats)

    gold_parts = pl.kernel(
        _gold_sc_body,
        out_type=jax.ShapeDtypeStruct((32, 16), jnp.float32),
        mesh=plsc.VectorSubcoreMesh(core_axis_name="c", subcore_axis_name="s"),
        compiler_params=pltpu.CompilerParams(needs_layout_passes=False),
            scratch_types=[pltpu.VMEM((_B,), jnp.int32),
                       pltpu.VMEM((_T * _T,), jnp.float32),
                       pltpu.VMEM((_HALF * _T,), jnp.float32),
                       pltpu.VMEM((_HALF,), jnp.int32),
                       pltpu.VMEM((_HALF,), jnp.int32),
                       pltpu.VMEM((16,), jnp.float32)],
    )(seq_lengths, transitions.reshape(_T * _T), feats.reshape(_B * _L * _T),
      tags.reshape(_B * _L), tags_prev.reshape(_B * _L))

    return (jnp.sum(fs) - jnp.sum(gold_parts)) / _B


# speedup vs baseline: 1.0403x; 1.0403x over previous
import jax
import jax.numpy as jnp
from jax.experimental import pallas as pl
from jax.experimental.pallas import tpu as pltpu
from jax.experimental.pallas import tpu_sc as plsc

_TAGSET = 48
_T = 50
_START = 48
_STOP = 49
_B = 16
_L = 2048
_CHUNK = 256
_RENORM = 4


def _forward_body(sl_ref, transT_ref, trans_ref, featsF_ref, featsR_ref,
                  tagsF_ref, tagsB_ref, out_ref, out2_ref,
                  aF_ref, cF_ref, aB_ref, cB_ref, egF_ref, egB_ref):
    pid = pl.program_id(0)
    trans = trans_ref[...]
    transT = transT_ref[...]
    mrow = jnp.max(transT, axis=0, keepdims=True)
    eT = jnp.exp(transT - mrow)
    mcolB = jnp.max(trans, axis=0, keepdims=True)
    EB = jnp.exp(trans - mcolB)
    emcB = jnp.exp(mcolB)

    @pl.when(pid == 0)
    def _init():
        lane = jax.lax.broadcasted_iota(jnp.int32, (_B, _T), 1)
        aF_ref[...] = jnp.where(lane == _START, 1.0, 0.0)
        cF_ref[...] = jnp.zeros((_B, 1), jnp.float32)
        srow = trans[_STOP:_STOP + 1, :]
        m0 = jnp.max(srow, axis=1, keepdims=True)
        aB_ref[...] = jnp.broadcast_to(jnp.exp(srow - m0), (_B, _T))
        cB_ref[...] = jnp.broadcast_to(m0, (_B, 1))
        out2_ref[...] = jnp.zeros((1, 1), jnp.float32)

    egF_ref[...] = jnp.exp(jnp.transpose(featsF_ref[...], (1, 0, 2)) + mrow)
    egB_ref[...] = jnp.exp(jnp.transpose(featsR_ref[...], (1, 0, 2)))
    sl = sl_ref[...]

    def _emit_part(f, tg, base_t):
        tag3 = tg[:, :, None]
        lane3 = jax.lax.broadcasted_iota(jnp.int32, (_B, _CHUNK, _T), 2)
        t3 = jax.lax.broadcasted_iota(jnp.int32, (_B, _CHUNK, 1), 1) + base_t
        keep = (lane3 == tag3) & (t3 < sl[:, :, None])
        return jnp.sum(jnp.where(keep, f, 0.0))

    out2_ref[...] = (out2_ref[...]
                     + _emit_part(featsF_ref[...], tagsF_ref[...],
                                  pid * _CHUNK)
                     + _emit_part(featsR_ref[...], tagsB_ref[...],
                                  _L - (pid + 1) * _CHUNK))

    def group(g, st):
        aF, cF, aB, cB = st
        for j in range(_RENORM):
            i = g * _RENORM + j
            tF = pid * _CHUNK + i
            tB = _L - 1 - tF
            sF = jax.lax.dot_general(aF, eT, (((1,), (0,)), ((), ())),
                                     preferred_element_type=jnp.float32)
            sF = sF * egF_ref[i]
            aF = jnp.where(tF < sl, sF, aF)
            sB = jax.lax.dot_general(aB * egB_ref[_CHUNK - 1 - i], EB,
                                     (((1,), (0,)), ((), ())),
                                     preferred_element_type=jnp.float32)
            sB = sB * emcB
            aB = jnp.where(tB < sl, sB, aB)
        mF = jnp.max(aF, axis=1, keepdims=True)
        mB = jnp.max(aB, axis=1, keepdims=True)
        return (aF * (1.0 / mF), cF + jnp.log(mF),
                aB * (1.0 / mB), cB + jnp.log(mB))

    aF, cF, aB, cB = jax.lax.fori_loop(
        0, _CHUNK // _RENORM, group,
        (aF_ref[...], cF_ref[...], aB_ref[...], cB_ref[...]))
    aF_ref[...] = aF
    cF_ref[...] = cF
    aB_ref[...] = aB
    cB_ref[...] = cB

    @pl.when(pid == pl.num_programs(0) - 1)
    def _final():
        d = jnp.sum(aF * aB, axis=1, keepdims=True)
        out_ref[...] = cF + cB + jnp.log(d)


_HALF = _L // 2


def _gold_sc_body(sl_hbm, trans_hbm, tags_hbm, tagsp_hbm, out_hbm,
                  sl_v, trans_v, tags_v, tagsp_v, acc_v):
    c = jax.lax.axis_index("c")
    s = jax.lax.axis_index("s")
    wid = s * 2 + c
    b = wid // 2
    h = wid % 2
    t0 = h * _HALF
    pltpu.sync_copy(sl_hbm, sl_v)
    pltpu.sync_copy(trans_hbm, trans_v)
    pltpu.sync_copy(tags_hbm.at[pl.ds(b * _L + t0, _HALF)], tags_v)
    pltpu.sync_copy(tagsp_hbm.at[pl.ds(b * _L + t0, _HALF)], tagsp_v)
    b_vec = jnp.zeros((16,), jnp.int32) + b
    slv = plsc.load_gather(sl_v, [b_vec])
    stop_vec = jnp.full((16,), _STOP, jnp.int32)

    def chunk(i, acc):
        tl = jax.lax.iota(jnp.int32, 16) + i * 16
        tn = tags_v[pl.ds(i * 16, 16)]
        tp = tagsp_v[pl.ds(i * 16, 16)]
        g = plsc.load_gather(trans_v, [tn * _T + tp])
        st = plsc.load_gather(trans_v, [stop_vec * _T + tn])
        tg = tl + t0
        val = jnp.where(tg < slv, g, 0.0)
        val = val + jnp.where(tg == slv - 1, st, 0.0)
        return acc + val

    acc = jax.lax.fori_loop(0, _HALF // 16, chunk,
                            jnp.zeros((16,), jnp.float32))
    acc_v[...] = acc
    pltpu.sync_copy(acc_v, out_hbm.at[wid])


def kernel(feats, tags, seq_lengths, transitions):
    transT = jnp.transpose(transitions, (1, 0))
    sl_col = seq_lengths.reshape(_B, 1)
    tags_prev = jnp.concatenate(
        [jnp.full((_B, 1), _START, dtype=tags.dtype), tags[:, :-1]], axis=1)

    half = _L // 2
    n_blocks = half // _CHUNK
    nb_total = _L // _CHUNK
    fs, emit_sum = pl.pallas_call(
        _forward_body,
        grid=(n_blocks,),
        in_specs=[
            pl.BlockSpec((_B, 1), lambda i: (0, 0)),
            pl.BlockSpec((_T, _T), lambda i: (0, 0)),
            pl.BlockSpec((_T, _T), lambda i: (0, 0)),
            pl.BlockSpec((_B, _CHUNK, _T), lambda i: (0, i, 0)),
            pl.BlockSpec((_B, _CHUNK, _T), lambda i: (0, nb_total - 1 - i, 0)),
            pl.BlockSpec((_B, _CHUNK), lambda i: (0, i)),
            pl.BlockSpec((_B, _CHUNK), lambda i: (0, nb_total - 1 - i)),
        ],
        out_specs=[pl.BlockSpec((_B, 1), lambda i: (0, 0)),
                   pl.BlockSpec((1, 1), lambda i: (0, 0))],
        out_shape=[jax.ShapeDtypeStruct((_B, 1), jnp.float32),
                   jax.ShapeDtypeStruct((1, 1), jnp.float32)],
        scratch_shapes=[pltpu.VMEM((_B, _T), jnp.float32),
                        pltpu.VMEM((_B, 1), jnp.float32),
                        pltpu.VMEM((_B, _T), jnp.float32),
                        pltpu.VMEM((_B, 1), jnp.float32),
                        pltpu.VMEM((_CHUNK, _B, _T), jnp.float32),
                        pltpu.VMEM((_CHUNK, _B, _T), jnp.float32)],
        compiler_params=pltpu.CompilerParams(
            dimension_semantics=("arbitrary",)),
    )(sl_col, transT, transitions, feats, feats, tags, tags)

    gold_parts = pl.kernel(
        _gold_sc_body,
        out_type=jax.ShapeDtypeStruct((32, 16), jnp.float32),
        mesh=plsc.VectorSubcoreMesh(core_axis_name="c", subcore_axis_name="s"),
        compiler_params=pltpu.CompilerParams(needs_layout_passes=False),
        scratch_types=[pltpu.VMEM((_B,), jnp.int32),
                       pltpu.VMEM((_T * _T,), jnp.float32),
                       pltpu.VMEM((_HALF,), jnp.int32),
                       pltpu.VMEM((_HALF,), jnp.int32),
                       pltpu.VMEM((16,), jnp.float32)],
    )(seq_lengths, transitions.reshape(_T * _T),
      tags.reshape(_B * _L), tags_prev.reshape(_B * _L))

    return (jnp.sum(fs) - emit_sum[0, 0] - jnp.sum(gold_parts)) / _B


# speedup vs baseline: 1.1065x; 1.0636x over previous
import jax
import jax.numpy as jnp
from jax.experimental import pallas as pl
from jax.experimental.pallas import tpu as pltpu
from jax.experimental.pallas import tpu_sc as plsc

_TAGSET = 48
_T = 50
_START = 48
_STOP = 49
_B = 16
_L = 2048
_CHUNK = 256
_RENORM = 4


def _forward_body(sl_ref, slr_ref, transT_ref, trans_ref, featsF_ref,
                  featsR_ref, tagsF_ref, tagsB_ref, out_ref, out2_ref,
                  aF_ref, cF_ref, aB_ref, cB_ref, rF_ref, rB_ref):
    pid = pl.program_id(0)
    trans = trans_ref[...]
    transT = transT_ref[...]
    mrow = jnp.max(transT, axis=0, keepdims=True)
    eT = jnp.exp(transT - mrow)
    mcolB = jnp.max(trans, axis=0, keepdims=True)
    EB = jnp.exp(trans - mcolB)
    emcB = jnp.exp(mcolB)

    @pl.when(pid == 0)
    def _init():
        lane = jax.lax.broadcasted_iota(jnp.int32, (_B, _T), 1)
        aF_ref[...] = jnp.where(lane == _START, 1.0, 0.0)
        cF_ref[...] = jnp.zeros((_B, 1), jnp.float32)
        srow = trans[_STOP:_STOP + 1, :]
        m0 = jnp.max(srow, axis=1, keepdims=True)
        aB_ref[...] = jnp.broadcast_to(jnp.exp(srow - m0), (_B, _T))
        cB_ref[...] = jnp.broadcast_to(m0, (_B, 1))
        rF_ref[...] = jnp.ones((_B, 1), jnp.float32)
        rB_ref[...] = jnp.ones((_B, 1), jnp.float32)
        out2_ref[...] = jnp.zeros((1, 1), jnp.float32)

    sl = sl_ref[...]

    def _emit_part(f, tg, base_t):
        tag3 = tg[:, :, None]
        lane3 = jax.lax.broadcasted_iota(jnp.int32, (_CHUNK, _B, _T), 2)
        t3 = jax.lax.broadcasted_iota(jnp.int32, (_CHUNK, _B, 1), 0) + base_t
        keep = (lane3 == tag3) & (t3 < slr_ref[...][:, :, None])
        return jnp.sum(jnp.where(keep, f, 0.0))

    out2_ref[...] = (out2_ref[...]
                     + _emit_part(featsF_ref[...], tagsF_ref[...],
                                  pid * _CHUNK)
                     + _emit_part(featsR_ref[...], tagsB_ref[...],
                                  _L - (pid + 1) * _CHUNK))

    def group(g, st):
        aF, cF, aB, cB, rF, rB = st
        for j in range(_RENORM):
            i = g * _RENORM + j
            tF = pid * _CHUNK + i
            tB = _L - 1 - tF
            egF = jnp.exp(featsF_ref[i] + mrow)
            egB = jnp.exp(featsR_ref[_CHUNK - 1 - i])
            sF = jax.lax.dot_general(aF, eT, (((1,), (0,)), ((), ())),
                                     preferred_element_type=jnp.float32)
            if j == 0:
                sF = sF * (egF * rF)
                aF = jnp.where(tF < sl, sF, aF * rF)
            else:
                sF = sF * egF
                aF = jnp.where(tF < sl, sF, aF)
            sB = jax.lax.dot_general(aB * egB, EB, (((1,), (0,)), ((), ())),
                                     preferred_element_type=jnp.float32)
            if j == 0:
                sB = sB * (emcB * rB)
                aB = jnp.where(tB < sl, sB, aB * rB)
            else:
                sB = sB * emcB
                aB = jnp.where(tB < sl, sB, aB)
        mF = jnp.max(aF, axis=1, keepdims=True)
        mB = jnp.max(aB, axis=1, keepdims=True)
        return (aF, cF + jnp.log(mF), aB, cB + jnp.log(mB),
                1.0 / mF, 1.0 / mB)

    aF, cF, aB, cB, rF, rB = jax.lax.fori_loop(
        0, _CHUNK // _RENORM, group,
        (aF_ref[...], cF_ref[...], aB_ref[...], cB_ref[...],
         rF_ref[...], rB_ref[...]))
    aF_ref[...] = aF
    cF_ref[...] = cF
    aB_ref[...] = aB
    cB_ref[...] = cB
    rF_ref[...] = rF
    rB_ref[...] = rB

    @pl.when(pid == pl.num_programs(0) - 1)
    def _final():
        d = jnp.sum((aF * rF) * (aB * rB), axis=1, keepdims=True)
        out_ref[...] = cF + cB + jnp.log(d)


_HALF = _L // 2


def _gold_sc_body(sl_hbm, trans_hbm, tags_hbm, tagsp_hbm, out_hbm,
                  sl_v, trans_v, tags_v, tagsp_v, acc_v):
    c = jax.lax.axis_index("c")
    s = jax.lax.axis_index("s")
    wid = s * 2 + c
    b = wid // 2
    h = wid % 2
    t0 = h * _HALF
    pltpu.sync_copy(sl_hbm, sl_v)
    pltpu.sync_copy(trans_hbm, trans_v)
    pltpu.sync_copy(tags_hbm.at[pl.ds(b * _L + t0, _HALF)], tags_v)
    pltpu.sync_copy(tagsp_hbm.at[pl.ds(b * _L + t0, _HALF)], tagsp_v)
    b_vec = jnp.zeros((16,), jnp.int32) + b
    slv = plsc.load_gather(sl_v, [b_vec])
    stop_vec = jnp.full((16,), _STOP, jnp.int32)

    def chunk(i, acc):
        tl = jax.lax.iota(jnp.int32, 16) + i * 16
        tn = tags_v[pl.ds(i * 16, 16)]
        tp = tagsp_v[pl.ds(i * 16, 16)]
        g = plsc.load_gather(trans_v, [tn * _T + tp])
        st = plsc.load_gather(trans_v, [stop_vec * _T + tn])
        tg = tl + t0
        val = jnp.where(tg < slv, g, 0.0)
        val = val + jnp.where(tg == slv - 1, st, 0.0)
        return acc + val

    acc = jax.lax.fori_loop(0, _HALF // 16, chunk,
                            jnp.zeros((16,), jnp.float32))
    acc_v[...] = acc
    pltpu.sync_copy(acc_v, out_hbm.at[wid])


def kernel(feats, tags, seq_lengths, transitions):
    transT = jnp.transpose(transitions, (1, 0))
    sl_col = seq_lengths.reshape(_B, 1)
    sl_row = seq_lengths.reshape(1, _B)
    featsT = jnp.transpose(feats, (1, 0, 2))
    tagsT = jnp.transpose(tags, (1, 0))
    tags_prev = jnp.concatenate(
        [jnp.full((_B, 1), _START, dtype=tags.dtype), tags[:, :-1]], axis=1)

    half = _L // 2
    n_blocks = half // _CHUNK
    nb_total = _L // _CHUNK
    fs, emit_sum = pl.pallas_call(
        _forward_body,
        grid=(n_blocks,),
        in_specs=[
            pl.BlockSpec((_B, 1), lambda i: (0, 0)),
            pl.BlockSpec((1, _B), lambda i: (0, 0)),
            pl.BlockSpec((_T, _T), lambda i: (0, 0)),
            pl.BlockSpec((_T, _T), lambda i: (0, 0)),
            pl.BlockSpec((_CHUNK, _B, _T), lambda i: (i, 0, 0)),
            pl.BlockSpec((_CHUNK, _B, _T), lambda i: (nb_total - 1 - i, 0, 0)),
            pl.BlockSpec((_CHUNK, _B), lambda i: (i, 0)),
            pl.BlockSpec((_CHUNK, _B), lambda i: (nb_total - 1 - i, 0)),
        ],
        out_specs=[pl.BlockSpec((_B, 1), lambda i: (0, 0)),
                   pl.BlockSpec((1, 1), lambda i: (0, 0))],
        out_shape=[jax.ShapeDtypeStruct((_B, 1), jnp.float32),
                   jax.ShapeDtypeStruct((1, 1), jnp.float32)],
        scratch_shapes=[pltpu.VMEM((_B, _T), jnp.float32),
                        pltpu.VMEM((_B, 1), jnp.float32),
                        pltpu.VMEM((_B, _T), jnp.float32),
                        pltpu.VMEM((_B, 1), jnp.float32),
                        pltpu.VMEM((_B, 1), jnp.float32),
                        pltpu.VMEM((_B, 1), jnp.float32)],
        compiler_params=pltpu.CompilerParams(
            dimension_semantics=("arbitrary",)),
    )(sl_col, sl_row, transT, transitions, featsT, featsT, tagsT, tagsT)

    gold_parts = pl.kernel(
        _gold_sc_body,
        out_type=jax.ShapeDtypeStruct((32, 16), jnp.float32),
        mesh=plsc.VectorSubcoreMesh(core_axis_name="c", subcore_axis_name="s"),
        compiler_params=pltpu.CompilerParams(needs_layout_passes=False),
        scratch_types=[pltpu.VMEM((_B,), jnp.int32),
                       pltpu.VMEM((_T * _T,), jnp.float32),
                       pltpu.VMEM((_HALF,), jnp.int32),
                       pltpu.VMEM((_HALF,), jnp.int32),
                       pltpu.VMEM((16,), jnp.float32)],
    )(seq_lengths, transitions.reshape(_T * _T),
      tags.reshape(_B * _L), tags_prev.reshape(_B * _L))

    return (jnp.sum(fs) - emit_sum[0, 0] - jnp.sum(gold_parts)) / _B
